# Initial kernel scaffold; baseline (speedup 1.0000x reference)
import functools

import jax
import jax.numpy as jnp
from jax import lax
from jax.experimental import pallas as pl
from jax.experimental.pallas import tpu as pltpu
from jax.experimental.pallas import tpu_sc as plsc

N = 320000
D = 128
S = 10000
SP = 10240
NC = 2
NS = 16
NW = NC * NS
CHUNK = 256
SUB = 128
UNITS = N // CHUNK
TRIPS = UNITS // NW
EXTRA = UNITS - TRIPS * NW
ZROWS = SP // NS


def _sc_segment_sums(rawret, labels2d):
    mesh = plsc.VectorSubcoreMesh(core_axis_name="c", subcore_axis_name="s")

    @functools.partial(
        pl.kernel,
        out_type=(
            jax.ShapeDtypeStruct((NC, SP, D), jnp.float32),
            jax.ShapeDtypeStruct((NC, SP, 16), jnp.float32),
        ),
        mesh=mesh,
        scratch_types=[
            pltpu.VMEM((CHUNK, D), jnp.float32),
            pltpu.VMEM((2, SUB), jnp.int32),
            pltpu.VMEM((SUB, 16), jnp.float32),
            pltpu.VMEM((128, D), jnp.float32),
            pltpu.VMEM((ZROWS, 16), jnp.float32),
            pltpu.VMEM_SHARED((SP, D), jnp.float32),
            pltpu.VMEM_SHARED((SP, 16), jnp.float32),
        ],
    )
    def seg_sum(raw_hbm, lbl_hbm, sums_hbm, cnt_hbm,
                rows_v, lbl_v, ones_v, sbuf, cbuf, acc_sh, cnt_sh):
        c = lax.axis_index("c")
        s = lax.axis_index("s")
        wid = c * NS + s

        zero16 = jnp.zeros((16,), jnp.float32)
        one16 = jnp.ones((16,), jnp.float32)

        @pl.loop(0, SUB)
        def _(i):
            ones_v[i, :] = one16

        @pl.loop(0, 128)
        def _(i):
            @pl.loop(0, D // 16)
            def _(j):
                sbuf[i, pl.ds(j * 16, 16)] = zero16

        @pl.loop(0, ZROWS)
        def _(i):
            cbuf[i, :] = zero16

        zrow = s * ZROWS
        for b in range(ZROWS // 128):
            pltpu.sync_copy(sbuf, acc_sh.at[pl.ds(zrow + b * 128, 128)])
        pltpu.sync_copy(cbuf, cnt_sh.at[pl.ds(zrow, ZROWS)])
        plsc.subcore_barrier()

        def do_chunk(u):
            pltpu.sync_copy(raw_hbm.at[pl.ds(u * CHUNK, CHUNK)], rows_v)
            pltpu.sync_copy(lbl_hbm.at[pl.ds(u * 2, 2)], lbl_v)
            for j in range(CHUNK // SUB):
                pltpu.sync_copy(rows_v.at[pl.ds(j * SUB, SUB)],
                                acc_sh.at[lbl_v.at[j]], add=True)
                pltpu.sync_copy(ones_v, cnt_sh.at[lbl_v.at[j]], add=True)

        @pl.loop(0, TRIPS)
        def _(t):
            do_chunk(TRIPS * wid + t)

        @pl.when(wid < EXTRA)
        def _():
            do_chunk(TRIPS * NW + wid)

        plsc.subcore_barrier()

        for b in range(ZROWS // 128):
            pltpu.sync_copy(acc_sh.at[pl.ds(zrow + b * 128, 128)], sbuf)
            pltpu.sync_copy(sbuf, sums_hbm.at[c].at[pl.ds(zrow + b * 128, 128)])
        pltpu.sync_copy(cnt_sh.at[pl.ds(zrow, ZROWS)], cbuf)
        pltpu.sync_copy(cbuf, cnt_hbm.at[c].at[pl.ds(zrow, ZROWS)])

    return seg_sum(rawret, labels2d)


def _combine(sums, cnts):
    BS = 1250

    def body(s_ref, c_ref, o_ref):
        total = s_ref[0] + s_ref[1]
        cnt = jnp.maximum(c_ref[0] + c_ref[1], 1.0)
        o_ref[...] = total / cnt[:, 0:1]

    return pl.pallas_call(
        body,
        grid=(S // BS,),
        in_specs=[
            pl.BlockSpec((NC, BS, D), lambda i: (0, i, 0)),
            pl.BlockSpec((NC, BS, 16), lambda i: (0, i, 0)),
        ],
        out_specs=pl.BlockSpec((BS, D), lambda i: (i, 0)),
        out_shape=jax.ShapeDtypeStruct((S, D), jnp.float32),
    )(sums, cnts)


def kernel(rawret, labels):
    labels2d = labels.reshape(N // 128, 128)
    sums, cnts = _sc_segment_sums(rawret, labels2d)
    return _combine(sums, cnts)


# speedup vs baseline: 5.3297x; 5.3297x over previous
import functools

import jax
import jax.numpy as jnp
from jax import lax
from jax.experimental import pallas as pl
from jax.experimental.pallas import tpu as pltpu
from jax.experimental.pallas import tpu_sc as plsc

N = 320000
D = 128
S = 10000
SP = 10240
NC = 2
NS = 16
DC = D // NC
CHUNK = 256
SUB = 128
UNITS = N // CHUNK
TRIPS = UNITS // NS
EXTRA = UNITS - TRIPS * NS
ZROWS = SP // NS


def _sc_segment_sums(rawret, labels2d):
    mesh = plsc.VectorSubcoreMesh(core_axis_name="c", subcore_axis_name="s")

    @functools.partial(
        pl.kernel,
        out_type=(
            jax.ShapeDtypeStruct((SP, D), jnp.float32),
            jax.ShapeDtypeStruct((SP, 16), jnp.float32),
        ),
        mesh=mesh,
        compiler_params=pltpu.CompilerParams(use_tc_tiling_on_sc=False),
        scratch_types=[
            pltpu.VMEM((CHUNK, DC), jnp.float32),
            pltpu.VMEM((2, SUB), jnp.int32),
            pltpu.VMEM((SUB, 16), jnp.float32),
            pltpu.VMEM((128, DC), jnp.float32),
            pltpu.VMEM((ZROWS, 16), jnp.float32),
            pltpu.VMEM_SHARED((SP, DC), jnp.float32),
            pltpu.VMEM_SHARED((SP, 16), jnp.float32),
        ],
    )
    def seg_sum(raw_hbm, lbl_hbm, sums_hbm, cnt_hbm,
                rows_v, lbl_v, ones_v, sbuf, cbuf, acc_sh, cnt_sh):
        c = lax.axis_index("c")
        s = lax.axis_index("s")
        col0 = c * DC

        zero16 = jnp.zeros((16,), jnp.float32)
        one16 = jnp.ones((16,), jnp.float32)

        @pl.loop(0, SUB)
        def _(i):
            ones_v[i, :] = one16

        @pl.loop(0, 128)
        def _(i):
            @pl.loop(0, DC // 16)
            def _(j):
                sbuf[i, pl.ds(j * 16, 16)] = zero16

        @pl.loop(0, ZROWS)
        def _(i):
            cbuf[i, :] = zero16

        zrow = s * ZROWS
        for b in range(ZROWS // 128):
            pltpu.sync_copy(sbuf, acc_sh.at[pl.ds(zrow + b * 128, 128)])
        pltpu.sync_copy(cbuf, cnt_sh.at[pl.ds(zrow, ZROWS)])
        plsc.subcore_barrier()

        def do_chunk(u):
            pltpu.sync_copy(
                raw_hbm.at[pl.ds(u * CHUNK, CHUNK), pl.ds(col0, DC)], rows_v)
            pltpu.sync_copy(lbl_hbm.at[pl.ds(u * 2, 2)], lbl_v)
            for j in range(CHUNK // SUB):
                pltpu.sync_copy(rows_v.at[pl.ds(j * SUB, SUB)],
                                acc_sh.at[lbl_v.at[j]], add=True)
                pltpu.sync_copy(ones_v, cnt_sh.at[lbl_v.at[j]], add=True)

        @pl.loop(0, TRIPS)
        def _(t):
            do_chunk(TRIPS * s + t)

        @pl.when(s < EXTRA)
        def _():
            do_chunk(TRIPS * NS + s)

        plsc.subcore_barrier()

        for b in range(ZROWS // 128):
            pltpu.sync_copy(acc_sh.at[pl.ds(zrow + b * 128, 128)], sbuf)
            pltpu.sync_copy(
                sbuf,
                sums_hbm.at[pl.ds(zrow + b * 128, 128), pl.ds(col0, DC)])

        @pl.when(c == 0)
        def _():
            pltpu.sync_copy(cnt_sh.at[pl.ds(zrow, ZROWS)], cbuf)
            pltpu.sync_copy(cbuf, cnt_hbm.at[pl.ds(zrow, ZROWS)])

    return seg_sum(rawret, labels2d)


def _combine(sums, cnts):
    BS = 2000

    def body(s_ref, c_ref, o_ref):
        cnt = jnp.maximum(c_ref[...], 1.0)
        o_ref[...] = s_ref[...] / cnt[:, 0:1]

    return pl.pallas_call(
        body,
        grid=(S // BS,),
        in_specs=[
            pl.BlockSpec((BS, D), lambda i: (i, 0)),
            pl.BlockSpec((BS, 16), lambda i: (i, 0)),
        ],
        out_specs=pl.BlockSpec((BS, D), lambda i: (i, 0)),
        out_shape=jax.ShapeDtypeStruct((S, D), jnp.float32),
    )(sums, cnts)


def kernel(rawret, labels):
    labels2d = labels.reshape(N // 128, 128)
    sums, cnts = _sc_segment_sums(rawret, labels2d)
    return _combine(sums, cnts)


# speedup vs baseline: 8.5137x; 1.5974x over previous
import functools

import jax
import jax.numpy as jnp
from jax import lax
from jax.experimental import pallas as pl
from jax.experimental.pallas import tpu as pltpu
from jax.experimental.pallas import tpu_sc as plsc

N = 320000
D = 128
S = 10000
SP = 10240
NC = 2
NS = 16
DC = D // NC
CHUNK = 256
SUB = 128
UNITS = N // CHUNK
TRIPS = UNITS // NS
EXTRA = UNITS - TRIPS * NS
ZROWS = SP // NS


def _sc_segment_sums(rawret, labels2d):
    mesh = plsc.VectorSubcoreMesh(core_axis_name="c", subcore_axis_name="s")

    @functools.partial(
        pl.kernel,
        out_type=(
            jax.ShapeDtypeStruct((SP, D), jnp.float32),
            jax.ShapeDtypeStruct((NC, SP, 16), jnp.float32),
        ),
        mesh=mesh,
        compiler_params=pltpu.CompilerParams(use_tc_tiling_on_sc=False),
        scratch_types=[
            pltpu.VMEM((2, CHUNK, DC), jnp.float32),
            pltpu.VMEM((2, 2, SUB), jnp.int32),
            pltpu.VMEM((SUB, 16), jnp.float32),
            pltpu.VMEM((128, DC), jnp.float32),
            pltpu.VMEM((ZROWS, 16), jnp.float32),
            pltpu.VMEM_SHARED((SP, DC), jnp.float32),
            pltpu.VMEM_SHARED((SP, 16), jnp.float32),
            pltpu.SemaphoreType.DMA,
            pltpu.SemaphoreType.DMA,
        ],
    )
    def seg_sum(raw_hbm, lbl_hbm, sums_hbm, cnt_hbm,
                rows_v, lbl_v, ones_v, sbuf, cbuf, acc_sh, cnt_sh,
                sem0, sem1):
        c = lax.axis_index("c")
        s = lax.axis_index("s")
        col0 = c * DC
        sems = (sem0, sem1)

        zero16 = jnp.zeros((16,), jnp.float32)
        one16 = jnp.ones((16,), jnp.float32)

        @pl.loop(0, SUB)
        def _(i):
            ones_v[i, :] = one16

        @pl.loop(0, 128)
        def _(i):
            @pl.loop(0, DC // 16)
            def _(j):
                sbuf[i, pl.ds(j * 16, 16)] = zero16

        @pl.loop(0, ZROWS)
        def _(i):
            cbuf[i, :] = zero16

        zrow = s * ZROWS
        for b in range(ZROWS // 128):
            pltpu.sync_copy(sbuf, acc_sh.at[pl.ds(zrow + b * 128, 128)])
        pltpu.sync_copy(cbuf, cnt_sh.at[pl.ds(zrow, ZROWS)])
        plsc.subcore_barrier()

        my_first = TRIPS * s
        do_counts = (c == 0) == (s < NS // 2)

        def rows_src(u):
            return raw_hbm.at[pl.ds(u * CHUNK, CHUNK), pl.ds(col0, DC)]

        def lbl_src(u):
            return lbl_hbm.at[pl.ds(u * 2, 2)]

        def dma_in(u, b):
            pltpu.async_copy(rows_src(u), rows_v.at[b], sems[b])
            pltpu.async_copy(lbl_src(u), lbl_v.at[b], sems[b])

        def dma_wait(u, b):
            pltpu.make_async_copy(rows_src(u), rows_v.at[b], sems[b]).wait()
            pltpu.make_async_copy(lbl_src(u), lbl_v.at[b], sems[b]).wait()

        def scatter(b, count_pred):
            for j in range(CHUNK // SUB):
                pltpu.sync_copy(rows_v.at[b].at[pl.ds(j * SUB, SUB)],
                                acc_sh.at[lbl_v.at[b].at[j]], add=True)

            @pl.when(count_pred)
            def _():
                for j in range(CHUNK // SUB):
                    pltpu.sync_copy(ones_v, cnt_sh.at[lbl_v.at[b].at[j]],
                                    add=True)

        dma_in(my_first, 0)

        @pl.loop(0, TRIPS // 2)
        def _(o):
            for b in range(2):
                t = 2 * o + b
                u = my_first + t
                dma_wait(u, b)

                @pl.when(t + 1 < TRIPS)
                def _():
                    dma_in(u + 1, 1 - b)

                scatter(b, do_counts)

        @pl.when(s < EXTRA)
        def _():
            u = TRIPS * NS + s
            pltpu.sync_copy(rows_src(u), rows_v.at[0])
            pltpu.sync_copy(lbl_src(u), lbl_v.at[0])
            scatter(0, c == 1)

        plsc.subcore_barrier()

        for b in range(ZROWS // 128):
            pltpu.sync_copy(acc_sh.at[pl.ds(zrow + b * 128, 128)], sbuf)
            pltpu.sync_copy(
                sbuf,
                sums_hbm.at[pl.ds(zrow + b * 128, 128), pl.ds(col0, DC)])
        pltpu.sync_copy(cnt_sh.at[pl.ds(zrow, ZROWS)], cbuf)
        pltpu.sync_copy(cbuf, cnt_hbm.at[c].at[pl.ds(zrow, ZROWS)])

    return seg_sum(rawret, labels2d)


def _combine(sums, cnts):
    BS = 2000

    def body(s_ref, c_ref, o_ref):
        cnt = jnp.maximum(c_ref[0] + c_ref[1], 1.0)
        o_ref[...] = s_ref[...] / cnt[:, 0:1]

    return pl.pallas_call(
        body,
        grid=(S // BS,),
        in_specs=[
            pl.BlockSpec((BS, D), lambda i: (i, 0)),
            pl.BlockSpec((NC, BS, 16), lambda i: (0, i, 0)),
        ],
        out_specs=pl.BlockSpec((BS, D), lambda i: (i, 0)),
        out_shape=jax.ShapeDtypeStruct((S, D), jnp.float32),
    )(sums, cnts)


def kernel(rawret, labels):
    labels2d = labels.reshape(N // 128, 128)
    sums, cnts = _sc_segment_sums(rawret, labels2d)
    return _combine(sums, cnts)
